# baseline (speedup 1.0000x reference)
import jax
import jax.numpy as jnp
from jax.experimental import pallas as pl
from jax.experimental.pallas import tpu as pltpu

_B, _N, _C = 2, 4096, 128
_TN = 256
_NT = _N // _TN
_NS = float(_B * _N)
_EPS = 1e-5


def _mega_body(x_ref, adj_ref, w_ref, b_ref, g_ref, be_ref,
               out_ref, adjv, yv, h32, h2, st):
    l = pl.program_id(0)
    b = pl.program_id(1)
    i = pl.program_id(2)

    @pl.when(jnp.logical_and(l == 0, jnp.logical_and(b == 0, i == 0)))
    def _init():
        st[0:1, :] = jnp.zeros((1, _C), jnp.float32)
        st[1:2, :] = jnp.full((1, _C), _NS, jnp.float32)
        st[2:4, :] = jnp.zeros((2, _C), jnp.float32)

    @pl.when(jnp.logical_and(l >= 1, jnp.logical_and(b == 0, i == 0)))
    def _roll_stats():
        st[0:2, :] = st[2:4, :]
        st[2:4, :] = jnp.zeros((2, _C), jnp.float32)

    def _coeffs():
        s = st[0:1, :]
        ss = st[1:2, :]
        mean = s * (1.0 / _NS)
        var = ss * (1.0 / _NS) - mean * mean
        v = var + _EPS
        r0 = jax.lax.rsqrt(v)
        r0 = r0 * (1.5 - 0.5 * v * r0 * r0)
        scale = g_ref[0] * r0
        shift = be_ref[0] - mean * scale
        isl0 = l == 0
        scale = jnp.where(isl0, jnp.ones_like(scale), scale)
        shift = jnp.where(isl0, jnp.zeros_like(shift), shift)
        return scale, shift

    @pl.when(jnp.logical_and(l <= 2, i == 0))
    def _compute_h():
        scale, shift = _coeffs()

        @pl.when(l == 0)
        def _hx():
            h32[...] = jnp.dot(x_ref[0], w_ref[0],
                               preferred_element_type=jnp.float32)

        @pl.when(l >= 1)
        def _hy():
            yp = yv[b] * scale + shift
            h32[...] = jnp.dot(yp, w_ref[0],
                               preferred_element_type=jnp.float32)

        @pl.when(jnp.logical_and(l >= 1, b == 0))
        def _hsplit():
            hh = h32[...]
            hx16 = hh.astype(jnp.bfloat16)
            h2[:, : _C] = hx16
            h2[:, _C:] = (hh - hx16.astype(jnp.float32)).astype(jnp.bfloat16)

    def _epilogue(out):
        y = jnp.maximum(out + b_ref[0], 0.0)
        yv[b, pl.ds(i * _TN, _TN), :] = y
        st[2:3, :] += jnp.sum(y, axis=0, keepdims=True)
        st[3:4, :] += jnp.sum(y * y, axis=0, keepdims=True)

    f32_path = jnp.logical_or(l == 0, jnp.logical_and(l <= 2, b == 1))

    @pl.when(f32_path)
    def _f32_tile():
        adj_blk = adj_ref[0]
        out = jnp.dot(adj_blk, h32[...], preferred_element_type=jnp.float32)
        sub = adj_ref[0, :, pl.ds(i * _TN, _TN)]
        rr = jax.lax.broadcasted_iota(jnp.int32, (_TN, _TN), 0)
        cc = jax.lax.broadcasted_iota(jnp.int32, (_TN, _TN), 1)
        eye = rr == cc
        diag = jnp.sum(jnp.where(eye, sub, 0.0), axis=1)
        htile = h32[pl.ds(i * _TN, _TN), :]
        out = out + (1.0 - diag)[:, None] * htile

        @pl.when(jnp.logical_and(l == 0, b == 0))
        def _cache_bf16():
            adjv[pl.ds(i * _TN, _TN), :] = adj_blk.astype(jnp.bfloat16)
            adjv[pl.ds(i * _TN, _TN), pl.ds(i * _TN, _TN)] = jnp.where(
                eye, jnp.bfloat16(1.0), sub.astype(jnp.bfloat16))

        _epilogue(out)

    @pl.when(jnp.logical_and(l >= 1, jnp.logical_and(l <= 2, b == 0)))
    def _bf16_tile():
        out2 = jnp.dot(adjv[pl.ds(i * _TN, _TN), :], h2[...],
                       preferred_element_type=jnp.float32)
        _epilogue(out2[:, : _C] + out2[:, _C:])

    @pl.when(l == 3)
    def _final_bn():
        scale, shift = _coeffs()
        out_ref[0] = yv[b, pl.ds(i * _TN, _TN), :] * scale + shift


def _x_idx(l, b, i):
    return (jnp.where(l == 0, b, 1), 0, 0)


def _adj_idx(l, b, i):
    bb = jnp.where(l == 0, b, 1)
    use = jnp.logical_or(l == 0,
                         jnp.logical_and(l <= 2,
                                         jnp.logical_and(l >= 1, b == 1)))
    ii = jnp.where(use, i, _NT - 1)
    return (bb, ii, 0)


def _out_idx(l, b, i):
    return (jnp.where(l == 3, b, 0), jnp.where(l == 3, i, 0), 0)


def kernel(x, adj, W0, b0, gamma0, beta0, W1, b1, gamma1, beta1,
           W2, b2, gamma2, beta2):
    wstack = jnp.stack([W0, W1, W2])
    bstack = jnp.stack([b0, b1, b2]).reshape(3, 1, _C)
    gstack = jnp.stack([gamma0, gamma1, gamma2]).reshape(3, 1, _C)
    bestack = jnp.stack([beta0, beta1, beta2]).reshape(3, 1, _C)

    return pl.pallas_call(
        _mega_body,
        grid=(4, _B, _NT),
        in_specs=[
            pl.BlockSpec((1, _N, _C), _x_idx),
            pl.BlockSpec((1, _TN, _N), _adj_idx),
            pl.BlockSpec((1, _C, _C), lambda l, b, i: (jnp.minimum(l, 2), 0, 0)),
            pl.BlockSpec((1, 1, _C), lambda l, b, i: (jnp.minimum(l, 2), 0, 0)),
            pl.BlockSpec((1, 1, _C), lambda l, b, i: (jnp.maximum(l - 1, 0), 0, 0)),
            pl.BlockSpec((1, 1, _C), lambda l, b, i: (jnp.maximum(l - 1, 0), 0, 0)),
        ],
        out_specs=pl.BlockSpec((1, _TN, _C), _out_idx),
        out_shape=jax.ShapeDtypeStruct((_B, _N, _C), jnp.float32),
        scratch_shapes=[
            pltpu.VMEM((_N, _N), jnp.bfloat16),
            pltpu.VMEM((_B, _N, _C), jnp.float32),
            pltpu.VMEM((_N, _C), jnp.float32),
            pltpu.VMEM((_N, 2 * _C), jnp.bfloat16),
            pltpu.VMEM((4, _C), jnp.float32),
        ],
        compiler_params=pltpu.CompilerParams(
            dimension_semantics=("arbitrary", "arbitrary", "arbitrary")),
    )(x, adj, wstack, bstack, gstack, bestack)


# speedup vs baseline: 1.1903x; 1.1903x over previous
import jax
import jax.numpy as jnp
from jax.experimental import pallas as pl
from jax.experimental.pallas import tpu as pltpu

_B, _N, _C = 2, 4096, 128
_TN = 1024
_NT = _N // _TN
_NS = float(_B * _N)
_EPS = 1e-5


def _bn_coeffs(s, ss, gamma, beta):
    mean = s * (1.0 / _NS)
    var = ss * (1.0 / _NS) - mean * mean
    v = var + _EPS
    r = jax.lax.rsqrt(v)
    r = r * (1.5 - 0.5 * v * r * r)
    scale = gamma * r
    shift = beta - mean * scale
    return scale, shift


def _layer1_body(x_ref, adj_ref, w_ref, b_ref,
                 y_ref, sum_ref, ssq_ref, adj16_ref, h_ref):
    bi = pl.program_id(0)
    i = pl.program_id(1)

    @pl.when(jnp.logical_and(bi == 0, i == 0))
    def _init_stats():
        sum_ref[...] = jnp.zeros_like(sum_ref)
        ssq_ref[...] = jnp.zeros_like(ssq_ref)

    @pl.when(i == 0)
    def _compute_h():
        h_ref[...] = jnp.dot(x_ref[0], w_ref[...],
                             preferred_element_type=jnp.float32)

    adj_blk = adj_ref[0]
    out = jnp.dot(adj_blk, h_ref[...], preferred_element_type=jnp.float32)

    sub = adj_ref[0, :, pl.ds(i * _TN, _TN)]
    r = jax.lax.broadcasted_iota(jnp.int32, (_TN, _TN), 0)
    c = jax.lax.broadcasted_iota(jnp.int32, (_TN, _TN), 1)
    eye = r == c
    diag = jnp.sum(jnp.where(eye, sub, 0.0), axis=1)
    h_tile = h_ref[pl.ds(i * _TN, _TN), :]
    out = out + (1.0 - diag)[:, None] * h_tile

    adj16_ref[0] = adj_blk.astype(jnp.bfloat16)
    adj16_ref[0, :, pl.ds(i * _TN, _TN)] = jnp.where(
        eye, jnp.bfloat16(1.0), sub.astype(jnp.bfloat16))

    y = jnp.maximum(out + b_ref[...], 0.0)
    y_ref[0] = y
    sum_ref[...] += jnp.sum(y, axis=0, keepdims=True)
    ssq_ref[...] += jnp.sum(y * y, axis=0, keepdims=True)


def _fused23_body(yprev_ref, adj16_ref, s1_ref, ss1_ref,
                  g0_ref, be0_ref, w1_ref, b1_ref,
                  g1_ref, be1_ref, w2_ref, b2_ref,
                  y_ref, sum_ref, ssq_ref, h2_ref, yv, st):
    l = pl.program_id(0)
    b = pl.program_id(1)
    i = pl.program_id(2)

    @pl.when(jnp.logical_and(l == 0, jnp.logical_and(b == 0, i == 0)))
    def _init_stage0():
        st[...] = jnp.zeros_like(st)

    @pl.when(jnp.logical_and(l == 1, jnp.logical_and(b == 0, i == 0)))
    def _init_stage1():
        sum_ref[...] = jnp.zeros_like(sum_ref)
        ssq_ref[...] = jnp.zeros_like(ssq_ref)

    def _split(h):
        h16 = h.astype(jnp.bfloat16)
        h2_ref[:, : _C] = h16
        h2_ref[:, _C:] = (h - h16.astype(jnp.float32)).astype(jnp.bfloat16)

    @pl.when(jnp.logical_and(l == 0, i == 0))
    def _compute_h2():
        scale, shift = _bn_coeffs(s1_ref[...], ss1_ref[...],
                                  g0_ref[...], be0_ref[...])
        yp = yprev_ref[0] * scale + shift
        _split(jnp.dot(yp, w1_ref[...], preferred_element_type=jnp.float32))

    @pl.when(jnp.logical_and(l == 1, i == 0))
    def _compute_h3():
        scale, shift = _bn_coeffs(st[0:1, :], st[1:2, :],
                                  g1_ref[...], be1_ref[...])
        yp = yv[b] * scale + shift
        _split(jnp.dot(yp, w2_ref[...], preferred_element_type=jnp.float32))

    out2 = jnp.dot(adj16_ref[0], h2_ref[...],
                   preferred_element_type=jnp.float32)
    out = out2[:, : _C] + out2[:, _C:]
    bias = jnp.where(l == 0, b1_ref[...], b2_ref[...])
    y = jnp.maximum(out + bias, 0.0)
    y_ref[0] = y

    @pl.when(l == 0)
    def _keep_stage0():
        yv[b, pl.ds(i * _TN2, _TN2), :] = y
        st[0:1, :] += jnp.sum(y, axis=0, keepdims=True)
        st[1:2, :] += jnp.sum(y * y, axis=0, keepdims=True)

    @pl.when(l == 1)
    def _stats_stage1():
        sum_ref[...] += jnp.sum(y, axis=0, keepdims=True)
        ssq_ref[...] += jnp.sum(y * y, axis=0, keepdims=True)


def _bn_body(y_ref, s_ref, ss_ref, gamma_ref, beta_ref, out_ref):
    scale, shift = _bn_coeffs(s_ref[...], ss_ref[...],
                              gamma_ref[...], beta_ref[...])
    out_ref[0] = y_ref[0] * scale + shift


_vec_spec = pl.BlockSpec((1, _C), lambda b, i: (0, 0))
_act_spec = pl.BlockSpec((1, _N, _C), lambda bi, i: (bi, 0, 0))
_adj_spec = pl.BlockSpec((1, _TN, _N), lambda bi, i: (bi, i, 0))
_w_spec = pl.BlockSpec((_C, _C), lambda bi, i: (0, 0))
_y_spec = pl.BlockSpec((1, _TN, _C), lambda bi, i: (bi, i, 0))
_params = pltpu.CompilerParams(dimension_semantics=("arbitrary", "arbitrary"))
_stat_shape = jax.ShapeDtypeStruct((1, _C), jnp.float32)


def _layer1(x, adj, w, b):
    return pl.pallas_call(
        _layer1_body,
        grid=(_B, _NT),
        in_specs=[_act_spec, _adj_spec, _w_spec, _vec_spec],
        out_specs=[_y_spec, _vec_spec, _vec_spec, _adj_spec],
        out_shape=[
            jax.ShapeDtypeStruct((_B, _N, _C), jnp.float32),
            _stat_shape,
            _stat_shape,
            jax.ShapeDtypeStruct((_B, _N, _N), jnp.bfloat16),
        ],
        scratch_shapes=[pltpu.VMEM((_N, _C), jnp.float32)],
        compiler_params=_params,
    )(x, adj, w, b)


_TN2 = 1024
_NT2 = _N // _TN2
_vec3_spec = pl.BlockSpec((1, _C), lambda l, b, i: (0, 0))
_w3_spec = pl.BlockSpec((_C, _C), lambda l, b, i: (0, 0))
_act3_spec = pl.BlockSpec((1, _N, _C), lambda l, b, i: (b, 0, 0))
_adj3_spec = pl.BlockSpec((1, _TN2, _N), lambda l, b, i: (b, i, 0))
_y3_spec = pl.BlockSpec((1, _TN2, _C), lambda l, b, i: (b, i, 0))


def _fused23(y, adj16, s, ss, gamma0, beta0, w1, b1, gamma1, beta1, w2, b2):
    return pl.pallas_call(
        _fused23_body,
        grid=(2, _B, _NT2),
        in_specs=[_act3_spec, _adj3_spec, _vec3_spec, _vec3_spec,
                  _vec3_spec, _vec3_spec, _w3_spec, _vec3_spec,
                  _vec3_spec, _vec3_spec, _w3_spec, _vec3_spec],
        out_specs=[_y3_spec, _vec3_spec, _vec3_spec],
        out_shape=[
            jax.ShapeDtypeStruct((_B, _N, _C), jnp.float32),
            _stat_shape,
            _stat_shape,
        ],
        scratch_shapes=[pltpu.VMEM((_N, 2 * _C), jnp.bfloat16),
                        pltpu.VMEM((_B, _N, _C), jnp.float32),
                        pltpu.VMEM((2, _C), jnp.float32)],
        compiler_params=pltpu.CompilerParams(
            dimension_semantics=("arbitrary", "arbitrary", "arbitrary")),
    )(y, adj16, s, ss, gamma0, beta0, w1, b1, gamma1, beta1, w2, b2)


def _apply_bn(y, s, ss, gamma, beta):
    return pl.pallas_call(
        _bn_body,
        grid=(_B, _NT),
        in_specs=[_y_spec, _vec_spec, _vec_spec, _vec_spec, _vec_spec],
        out_specs=_y_spec,
        out_shape=jax.ShapeDtypeStruct((_B, _N, _C), jnp.float32),
        compiler_params=_params,
    )(y, s, ss, gamma, beta)


def kernel(x, adj, W0, b0, gamma0, beta0, W1, b1, gamma1, beta1,
           W2, b2, gamma2, beta2):
    r = lambda v: v.reshape(1, _C)

    y, s, ss, adj16 = _layer1(x, adj, W0, r(b0))
    y, s, ss = _fused23(y, adj16, s, ss, r(gamma0), r(beta0), W1, r(b1),
                        r(gamma1), r(beta1), W2, r(b2))
    return _apply_bn(y, s, ss, r(gamma2), r(beta2))
